# baseline (speedup 1.0000x reference)
import functools

import jax
import jax.numpy as jnp
from jax import lax
from jax.experimental import pallas as pl
from jax.experimental.pallas import tpu as pltpu
from jax.experimental.pallas import tpu_sc as plsc

_NC = 2
_NS = 16
_NW = _NC * _NS


def _sp(x):
    r = jnp.log1p(jnp.exp(x))
    return jnp.where(x > 20.0, x, r)


def _mm(a, w):
    return jnp.dot(a, w, preferred_element_type=jnp.float32)


def _bd(w, k):
    return jnp.kron(jnp.eye(k, dtype=w.dtype), w)


def _bt(b, k):
    return jnp.tile(b, k).reshape(1, -1)


def _sc_gather(h, src, dst):
    e = src.shape[0]
    per_w = e // _NW
    c = 200
    nchunks = per_w // c
    d = h.shape[1]
    mesh = plsc.VectorSubcoreMesh(core_axis_name="c", subcore_axis_name="s")

    dt = h.dtype

    def body(h_hbm, src_hbm, dst_hbm, hs_hbm, hd_hbm,
             sidx, didx, rows_a, rows_b, sem_a, sem_b):
        wid = lax.axis_index("s") * _NC + lax.axis_index("c")
        base = wid * per_w
        pltpu.sync_copy(src_hbm.at[pl.ds(base, per_w)], sidx)
        pltpu.sync_copy(dst_hbm.at[pl.ds(base, per_w)], didx)

        pltpu.async_copy(h_hbm.at[sidx.at[pl.ds(0, c)]], rows_a, sem_a)
        pltpu.async_copy(h_hbm.at[didx.at[pl.ds(0, c)]], rows_b, sem_b)

        def step(j, carry):
            off = j * c
            pltpu.make_async_copy(h_hbm.at[sidx.at[pl.ds(off, c)]], rows_a,
                                  sem_a).wait()
            pltpu.sync_copy(rows_a, hs_hbm.at[pl.ds(base + off, c)])

            @pl.when(j + 1 < nchunks)
            def _():
                pltpu.async_copy(h_hbm.at[sidx.at[pl.ds(off + c, c)]],
                                 rows_a, sem_a)

            pltpu.make_async_copy(h_hbm.at[didx.at[pl.ds(off, c)]], rows_b,
                                  sem_b).wait()
            pltpu.sync_copy(rows_b, hd_hbm.at[pl.ds(base + off, c)])

            @pl.when(j + 1 < nchunks)
            def _():
                pltpu.async_copy(h_hbm.at[didx.at[pl.ds(off + c, c)]],
                                 rows_b, sem_b)
            return carry
        lax.fori_loop(0, nchunks, step, 0)

    f = pl.kernel(
        body,
        out_type=[jax.ShapeDtypeStruct((e, d), dt),
                  jax.ShapeDtypeStruct((e, d), dt)],
        mesh=mesh,
        compiler_params=pltpu.CompilerParams(use_tc_tiling_on_sc=False),
        scratch_types=[pltpu.VMEM((per_w,), jnp.int32),
                       pltpu.VMEM((per_w,), jnp.int32),
                       pltpu.VMEM((c, d), dt),
                       pltpu.VMEM((c, d), dt),
                       pltpu.SemaphoreType.DMA,
                       pltpu.SemaphoreType.DMA],
    )
    return f(h, src, dst)


def _sc_scatter(msg, dst3, zeros, n):
    e = msg.shape[0]
    per_w = e // _NW
    kc, cb = dst3.shape[1], dst3.shape[2]
    rows_s = n // _NS
    d = msg.shape[1]
    mesh = plsc.VectorSubcoreMesh(core_axis_name="c", subcore_axis_name="s")

    def body(msg_hbm, dst3_hbm, zeros_hbm, out_hbm, idx_v, rows_a, rows_b,
             acc_sh, sem_a, sem_b):
        cid = lax.axis_index("c")
        sid = lax.axis_index("s")
        wid = sid * _NC + cid
        pltpu.sync_copy(zeros_hbm, acc_sh.at[pl.ds(sid * rows_s, rows_s)])
        pltpu.sync_copy(dst3_hbm.at[wid], idx_v)
        plsc.subcore_barrier()

        base = wid * per_w
        pltpu.async_copy(msg_hbm.at[pl.ds(base, cb)], rows_a, sem_a)

        def step(j2, carry):
            pltpu.async_copy(msg_hbm.at[pl.ds(base + (j2 + 1) * cb, cb)],
                             rows_b, sem_b)
            pltpu.make_async_copy(msg_hbm.at[pl.ds(base + j2 * cb, cb)],
                                  rows_a, sem_a).wait()
            pltpu.sync_copy(rows_a, acc_sh.at[idx_v.at[j2]], add=True)

            @pl.when(j2 + 2 < kc)
            def _():
                pltpu.async_copy(msg_hbm.at[pl.ds(base + (j2 + 2) * cb, cb)],
                                 rows_a, sem_a)

            pltpu.make_async_copy(msg_hbm.at[pl.ds(base + (j2 + 1) * cb, cb)],
                                  rows_b, sem_b).wait()
            pltpu.sync_copy(rows_b, acc_sh.at[idx_v.at[j2 + 1]], add=True)
            return carry
        lax.fori_loop(0, kc // 2, lambda i, c_: step(2 * i, c_), 0)
        plsc.subcore_barrier()
        pltpu.sync_copy(acc_sh.at[pl.ds(sid * rows_s, rows_s)],
                        out_hbm.at[cid, pl.ds(sid * rows_s, rows_s)])

    f = pl.kernel(
        body,
        out_type=jax.ShapeDtypeStruct((_NC, n, d), jnp.float32),
        mesh=mesh,
        compiler_params=pltpu.CompilerParams(use_tc_tiling_on_sc=False),
        scratch_types=[pltpu.VMEM((kc, cb), jnp.int32),
                       pltpu.VMEM((cb, d), jnp.float32),
                       pltpu.VMEM((cb, d), jnp.float32),
                       pltpu.VMEM_SHARED((n, d), jnp.float32),
                       pltpu.SemaphoreType.DMA,
                       pltpu.SemaphoreType.DMA],
    )
    return f(msg, dst3, zeros)


def _combine_body(h_ref, p_ref, q_ref, out_ref, outb_ref):
    hv = (h_ref[...] + (p_ref[0] + p_ref[1])
          + (q_ref[0] + q_ref[1]))
    out_ref[...] = hv
    outb_ref[...] = hv.astype(jnp.bfloat16)


def _combine(h2, p2, q2):
    n2, d = h2.shape
    bn = 1000
    return pl.pallas_call(
        _combine_body,
        grid=(n2 // bn,),
        in_specs=[pl.BlockSpec((bn, d), lambda i: (i, 0)),
                  pl.BlockSpec((2, bn, d), lambda i: (0, i, 0)),
                  pl.BlockSpec((2, bn, d), lambda i: (0, i, 0))],
        out_specs=[pl.BlockSpec((bn, d), lambda i: (i, 0)),
                   pl.BlockSpec((bn, d), lambda i: (i, 0))],
        out_shape=[jax.ShapeDtypeStruct((n2, d), jnp.float32),
                   jax.ShapeDtypeStruct((n2, d), jnp.bfloat16)],
    )(h2, p2, q2)


def _node_init_body(x_ref, nv_ref, aw1, ab1, aw2, ab2,
                    kw1, kb1, kw2, kb2, kw3, kb3, h_ref, hb_ref, ke_ref):
    t = _sp(_mm(x_ref[...], aw1[...]) + ab1[...])
    hv = _mm(t, aw2[...]) + ab2[...]
    h_ref[...] = hv
    hb_ref[...] = hv.astype(jnp.bfloat16)
    u = _sp(_mm(nv_ref[...], kw1[...]) + kb1[...])
    u = _sp(_mm(u, kw2[...]) + kb2[...])
    ke_ref[...] = _mm(u, kw3[...]) + kb3[...]


def _node_init(x2, nv2, fa_params, ke_params):
    n2 = x2.shape[0]
    bn = 1000
    ws = [y for (w, b) in fa_params for y in (_bd(w, 2), _bt(b, 2))]
    ws += [y for (w, b) in ke_params for y in (_bd(w, 2), _bt(b, 2))]
    w_specs = [pl.BlockSpec(w.shape, lambda i: (0, 0)) for w in ws]
    return pl.pallas_call(
        _node_init_body,
        grid=(n2 // bn,),
        in_specs=[
            pl.BlockSpec((bn, x2.shape[1]), lambda i: (i, 0)),
            pl.BlockSpec((bn, nv2.shape[1]), lambda i: (i, 0)),
        ] + w_specs,
        out_specs=[
            pl.BlockSpec((bn, 128), lambda i: (i, 0)),
            pl.BlockSpec((bn, 128), lambda i: (i, 0)),
            pl.BlockSpec((bn, 2), lambda i: (i, 0)),
        ],
        out_shape=[
            jax.ShapeDtypeStruct((n2, 128), jnp.float32),
            jax.ShapeDtypeStruct((n2, 128), jnp.bfloat16),
            jax.ShapeDtypeStruct((n2, 2), jnp.float32),
        ],
    )(x2, nv2, *ws)


_E_BLK2 = 3200


def _edge_first_body(hs, hd, eattr, bw1, bb1, bw2, bb2,
                     ew1, eb1, ew2, eb2, ew3, eb3,
                     va, vb, vb1, vw2, vb2, vw3, vb3, ea_out, msg_out):
    t0 = _sp(_mm(eattr[...], bw1[...]) + bb1[...])
    ea0 = _mm(t0, bw2[...]) + bb2[...]
    hsf = hs[...].astype(jnp.float32)
    hdf = hd[...].astype(jnp.float32)
    c2 = hsf * hdf
    t = _sp(_mm(c2, ew1[...]) + eb1[...])
    t = _sp(_mm(t, ew2[...]) + eb2[...])
    ea = _mm(t, ew3[...]) + eb3[...] + ea0
    ea_out[...] = ea
    u = _sp(_mm(hdf, va[...]) + _mm(ea, vb[...]) + vb1[...])
    u = _sp(_mm(u, vw2[...]) + vb2[...])
    msg_out[...] = _mm(u, vw3[...]) + vb3[...]


def _edge_mid_body(hs, hd, ea_in, ew1, eb1, ew2, eb2, ew3, eb3,
                   va, vb, vb1, vw2, vb2, vw3, vb3, ea_out, msg_out):
    hsf = hs[...].astype(jnp.float32)
    hdf = hd[...].astype(jnp.float32)
    c2 = hsf * hdf
    t = _sp(_mm(c2, ew1[...]) + eb1[...])
    t = _sp(_mm(t, ew2[...]) + eb2[...])
    ea = _mm(t, ew3[...]) + eb3[...] + ea_in[...]
    ea_out[...] = ea
    u = _sp(_mm(hdf, va[...]) + _mm(ea, vb[...]) + vb1[...])
    u = _sp(_mm(u, vw2[...]) + vb2[...])
    msg_out[...] = _mm(u, vw3[...]) + vb3[...]


def _edge_last_body(hs, hd, ea_in, ew1, eb1, ew2, eb2, ew3, eb3,
                    mw1, mb1, mw2, mb2, mw3, mb3, pe_out):
    hsf = hs[...].astype(jnp.float32)
    hdf = hd[...].astype(jnp.float32)
    c2 = hsf * hdf
    t = _sp(_mm(c2, ew1[...]) + eb1[...])
    t = _sp(_mm(t, ew2[...]) + eb2[...])
    ea = _mm(t, ew3[...]) + eb3[...] + ea_in[...]
    p = _sp(_mm(ea, mw1[...]) + mb1[...])
    p = _sp(_mm(p, mw2[...]) + mb2[...])
    pe_out[...] = jnp.transpose(_mm(p, mw3[...]) + mb3[...], (1, 0))


def _edge_step(kind, hs2, hd2, ea2, weight_list, ea_blk_off=0):
    e2 = hs2.shape[0]
    be = _E_BLK2
    body = {"first": _edge_first_body, "mid": _edge_mid_body,
            "last": _edge_last_body}[kind]
    w_specs = [pl.BlockSpec(w.shape, lambda i: (0, 0)) for w in weight_list]
    if kind == "last":
        out_specs = [pl.BlockSpec((2, be), lambda i: (0, i))]
        out_shape = [jax.ShapeDtypeStruct((2, e2), jnp.float32)]
    else:
        out_specs = [pl.BlockSpec((be, 64), lambda i: (i, 0)),
                     pl.BlockSpec((be, 128), lambda i: (i, 0))]
        out_shape = [jax.ShapeDtypeStruct((e2, 64), jnp.float32),
                     jax.ShapeDtypeStruct((e2, 128), jnp.float32)]
    return pl.pallas_call(
        body,
        grid=(e2 // be,),
        in_specs=[
            pl.BlockSpec((be, 128), lambda i: (i, 0)),
            pl.BlockSpec((be, 128), lambda i: (i, 0)),
            pl.BlockSpec((be, ea2.shape[1]), lambda i: (i + ea_blk_off, 0)),
        ] + w_specs,
        out_specs=out_specs,
        out_shape=out_shape,
    )(hs2, hd2, ea2, *weight_list)


def kernel(x, edge_attr, node_vel_emb, fa_params, fb_params, fe_params,
           fv_params, ke_params, mlp1_params, edge_index):
    n = x.shape[0]
    e = edge_attr.shape[0]
    src = edge_index[0]
    dst = edge_index[1]

    x2 = x.reshape(n // 2, 2 * x.shape[1])
    nv2 = node_vel_emb.reshape(n // 2, 2 * node_vel_emb.shape[1])
    eattr2 = edge_attr.reshape(e // 2, 2 * edge_attr.shape[1])

    h2, h2b, ke2 = _node_init(x2, nv2, fa_params, ke_params)

    (v1, b1), (v2, b2), (v3, b3) = fv_params
    fv_list = [_bd(v1[:64], 2), _bd(v1[64:], 2), _bt(b1, 2),
               _bd(v2, 2), _bt(b2, 2), _bd(v3, 2), _bt(b3, 2)]
    fb_list = [y for (w, b) in fb_params for y in (_bd(w, 2), _bt(b, 2))]
    fe_list = [y for (w, b) in fe_params for y in (_bd(w, 2), _bt(b, 2))]
    m1_list = [y for (w, b) in mlp1_params for y in (_bd(w, 2), _bt(b, 2))]

    eh = e // 2
    cb = 100
    srcs = [lax.slice(src, (0,), (eh,)), lax.slice(src, (eh,), (e,))]
    dsts = [lax.slice(dst, (0,), (eh,)), lax.slice(dst, (eh,), (e,))]
    dst3s = [d_.reshape(_NW, (eh // _NW) // cb, cb) for d_ in dsts]
    zeros = jnp.zeros((n // _NS, 64), jnp.float32)
    nblk_h = (eh // 2) // _E_BLK2

    eas = [eattr2, eattr2]
    ea_offs = [0, nblk_h]
    pes = [None, None]
    for step in range(3):
        h_lin = h2b.reshape(n, 64)
        gath = [_sc_gather(h_lin, srcs[i], dsts[i]) for i in range(2)]
        msgs = [None, None]
        for i in range(2):
            hs2 = gath[i][0].reshape(eh // 2, 128)
            hd2 = gath[i][1].reshape(eh // 2, 128)
            if step == 0:
                eas[i], msgs[i] = _edge_step(
                    "first", hs2, hd2, eas[i], fb_list + fe_list + fv_list,
                    ea_blk_off=ea_offs[i])
            elif step == 1:
                eas[i], msgs[i] = _edge_step("mid", hs2, hd2, eas[i],
                                             fe_list + fv_list)
            else:
                pes[i] = _edge_step("last", hs2, hd2, eas[i],
                                    fe_list + m1_list)[0]
        if step < 2:
            parts = [_sc_scatter(msgs[i].reshape(eh, 64), dst3s[i], zeros, n)
                     for i in range(2)]
            h2, h2b = _combine(h2, parts[0].reshape(2, n // 2, 128),
                               parts[1].reshape(2, n // 2, 128))

    pe = jnp.concatenate(
        [jnp.transpose(p_, (1, 0)).reshape(eh, 1) for p_ in pes], axis=0)
    ke_out = ke2.reshape(n, 1)
    return (pe, ke_out)


# speedup vs baseline: 1.7571x; 1.7571x over previous
import functools

import jax
import jax.numpy as jnp
from jax import lax
from jax.experimental import pallas as pl
from jax.experimental.pallas import tpu as pltpu
from jax.experimental.pallas import tpu_sc as plsc

_NC = 2
_NS = 16
_NW = _NC * _NS


def _sp(x):
    r = jnp.log1p(jnp.exp(x))
    return jnp.where(x > 20.0, x, r)


def _mm(a, w):
    return jnp.dot(a, w, preferred_element_type=jnp.float32)


def _bd(w, k):
    return jnp.kron(jnp.eye(k, dtype=w.dtype), w)


def _bt(b, k):
    return jnp.tile(b, k).reshape(1, -1)


def _sc_gather(h, src, dst):
    e = src.shape[0]
    per_w = e // _NW
    c = 200
    nchunks = per_w // c
    d = h.shape[1]
    mesh = plsc.VectorSubcoreMesh(core_axis_name="c", subcore_axis_name="s")

    def body(h_hbm, src_hbm, dst_hbm, hs_hbm, hd_hbm,
             sidx, didx, rows_a, rows_b, sem_a, sem_b):
        wid = lax.axis_index("s") * _NC + lax.axis_index("c")
        base = wid * per_w
        pltpu.sync_copy(src_hbm.at[pl.ds(base, per_w)], sidx)
        pltpu.sync_copy(dst_hbm.at[pl.ds(base, per_w)], didx)

        pltpu.async_copy(h_hbm.at[sidx.at[pl.ds(0, c)]], rows_a, sem_a)
        pltpu.async_copy(h_hbm.at[didx.at[pl.ds(0, c)]], rows_b, sem_b)

        def step(j, carry):
            off = j * c
            pltpu.make_async_copy(h_hbm.at[sidx.at[pl.ds(off, c)]], rows_a,
                                  sem_a).wait()
            pltpu.sync_copy(rows_a, hs_hbm.at[pl.ds(base + off, c)])

            @pl.when(j + 1 < nchunks)
            def _():
                pltpu.async_copy(h_hbm.at[sidx.at[pl.ds(off + c, c)]],
                                 rows_a, sem_a)

            pltpu.make_async_copy(h_hbm.at[didx.at[pl.ds(off, c)]], rows_b,
                                  sem_b).wait()
            pltpu.sync_copy(rows_b, hd_hbm.at[pl.ds(base + off, c)])

            @pl.when(j + 1 < nchunks)
            def _():
                pltpu.async_copy(h_hbm.at[didx.at[pl.ds(off + c, c)]],
                                 rows_b, sem_b)
            return carry
        lax.fori_loop(0, nchunks, step, 0)

    f = pl.kernel(
        body,
        out_type=[jax.ShapeDtypeStruct((e, d), jnp.float32),
                  jax.ShapeDtypeStruct((e, d), jnp.float32)],
        mesh=mesh,
        compiler_params=pltpu.CompilerParams(use_tc_tiling_on_sc=False),
        scratch_types=[pltpu.VMEM((per_w,), jnp.int32),
                       pltpu.VMEM((per_w,), jnp.int32),
                       pltpu.VMEM((c, d), jnp.float32),
                       pltpu.VMEM((c, d), jnp.float32),
                       pltpu.SemaphoreType.DMA,
                       pltpu.SemaphoreType.DMA],
    )
    return f(h, src, dst)


def _sc_scatter(msg, dst3, zeros, n):
    e = msg.shape[0]
    per_w = e // _NW
    kc, cb = dst3.shape[1], dst3.shape[2]
    rows_s = n // _NS
    d = msg.shape[1]
    mesh = plsc.VectorSubcoreMesh(core_axis_name="c", subcore_axis_name="s")

    def body(msg_hbm, dst3_hbm, zeros_hbm, out_hbm, idx_v, rows_a, rows_b,
             acc_sh, sem_a, sem_b):
        cid = lax.axis_index("c")
        sid = lax.axis_index("s")
        wid = sid * _NC + cid
        pltpu.sync_copy(zeros_hbm, acc_sh.at[pl.ds(sid * rows_s, rows_s)])
        pltpu.sync_copy(dst3_hbm.at[wid], idx_v)
        plsc.subcore_barrier()

        base = wid * per_w
        pltpu.async_copy(msg_hbm.at[pl.ds(base, cb)], rows_a, sem_a)

        def step(j2, carry):
            pltpu.async_copy(msg_hbm.at[pl.ds(base + (j2 + 1) * cb, cb)],
                             rows_b, sem_b)
            pltpu.make_async_copy(msg_hbm.at[pl.ds(base + j2 * cb, cb)],
                                  rows_a, sem_a).wait()
            pltpu.sync_copy(rows_a, acc_sh.at[idx_v.at[j2]], add=True)

            @pl.when(j2 + 2 < kc)
            def _():
                pltpu.async_copy(msg_hbm.at[pl.ds(base + (j2 + 2) * cb, cb)],
                                 rows_a, sem_a)

            pltpu.make_async_copy(msg_hbm.at[pl.ds(base + (j2 + 1) * cb, cb)],
                                  rows_b, sem_b).wait()
            pltpu.sync_copy(rows_b, acc_sh.at[idx_v.at[j2 + 1]], add=True)
            return carry
        lax.fori_loop(0, kc // 2, lambda i, c_: step(2 * i, c_), 0)
        plsc.subcore_barrier()
        pltpu.sync_copy(acc_sh.at[pl.ds(sid * rows_s, rows_s)],
                        out_hbm.at[cid, pl.ds(sid * rows_s, rows_s)])

    f = pl.kernel(
        body,
        out_type=jax.ShapeDtypeStruct((_NC, n, d), jnp.float32),
        mesh=mesh,
        compiler_params=pltpu.CompilerParams(use_tc_tiling_on_sc=False),
        scratch_types=[pltpu.VMEM((kc, cb), jnp.int32),
                       pltpu.VMEM((cb, d), jnp.float32),
                       pltpu.VMEM((cb, d), jnp.float32),
                       pltpu.VMEM_SHARED((n, d), jnp.float32),
                       pltpu.SemaphoreType.DMA,
                       pltpu.SemaphoreType.DMA],
    )
    return f(msg, dst3, zeros)


def _combine_body(h_ref, p_ref, q_ref, out_ref):
    out_ref[...] = (h_ref[...] + (p_ref[0] + p_ref[1])
                    + (q_ref[0] + q_ref[1]))


def _combine(h2, p2, q2):
    n2, d = h2.shape
    bn = 1000
    return pl.pallas_call(
        _combine_body,
        grid=(n2 // bn,),
        in_specs=[pl.BlockSpec((bn, d), lambda i: (i, 0)),
                  pl.BlockSpec((2, bn, d), lambda i: (0, i, 0)),
                  pl.BlockSpec((2, bn, d), lambda i: (0, i, 0))],
        out_specs=pl.BlockSpec((bn, d), lambda i: (i, 0)),
        out_shape=jax.ShapeDtypeStruct((n2, d), jnp.float32),
    )(h2, p2, q2)


def _node_init_body(x_ref, nv_ref, aw1, ab1, aw2, ab2,
                    kw1, kb1, kw2, kb2, kw3, kb3, h_ref, ke_ref):
    t = _sp(_mm(x_ref[...], aw1[...]) + ab1[...])
    h_ref[...] = _mm(t, aw2[...]) + ab2[...]
    u = _sp(_mm(nv_ref[...], kw1[...]) + kb1[...])
    u = _sp(_mm(u, kw2[...]) + kb2[...])
    ke_ref[...] = _mm(u, kw3[...]) + kb3[...]


def _node_init(x2, nv2, fa_params, ke_params):
    n2 = x2.shape[0]
    bn = 1000
    ws = [y for (w, b) in fa_params for y in (_bd(w, 2), _bt(b, 2))]
    ws += [y for (w, b) in ke_params for y in (_bd(w, 2), _bt(b, 2))]
    w_specs = [pl.BlockSpec(w.shape, lambda i: (0, 0)) for w in ws]
    return pl.pallas_call(
        _node_init_body,
        grid=(n2 // bn,),
        in_specs=[
            pl.BlockSpec((bn, x2.shape[1]), lambda i: (i, 0)),
            pl.BlockSpec((bn, nv2.shape[1]), lambda i: (i, 0)),
        ] + w_specs,
        out_specs=[
            pl.BlockSpec((bn, 128), lambda i: (i, 0)),
            pl.BlockSpec((bn, 2), lambda i: (i, 0)),
        ],
        out_shape=[
            jax.ShapeDtypeStruct((n2, 128), jnp.float32),
            jax.ShapeDtypeStruct((n2, 2), jnp.float32),
        ],
    )(x2, nv2, *ws)


_E_BLK2 = 3200


def _edge_first_body(hs, hd, eattr, bw1, bb1, bw2, bb2,
                     ew1, eb1, ew2, eb2, ew3, eb3,
                     va, vb, vb1, vw2, vb2, vw3, vb3, ea_out, msg_out):
    t0 = _sp(_mm(eattr[...], bw1[...]) + bb1[...])
    ea0 = _mm(t0, bw2[...]) + bb2[...]
    c2 = hs[...] * hd[...]
    t = _sp(_mm(c2, ew1[...]) + eb1[...])
    t = _sp(_mm(t, ew2[...]) + eb2[...])
    ea = _mm(t, ew3[...]) + eb3[...] + ea0
    ea_out[...] = ea
    u = _sp(_mm(hd[...], va[...]) + _mm(ea, vb[...]) + vb1[...])
    u = _sp(_mm(u, vw2[...]) + vb2[...])
    msg_out[...] = _mm(u, vw3[...]) + vb3[...]


def _edge_mid_body(hs, hd, ea_in, ew1, eb1, ew2, eb2, ew3, eb3,
                   va, vb, vb1, vw2, vb2, vw3, vb3, ea_out, msg_out):
    c2 = hs[...] * hd[...]
    t = _sp(_mm(c2, ew1[...]) + eb1[...])
    t = _sp(_mm(t, ew2[...]) + eb2[...])
    ea = _mm(t, ew3[...]) + eb3[...] + ea_in[...]
    ea_out[...] = ea
    u = _sp(_mm(hd[...], va[...]) + _mm(ea, vb[...]) + vb1[...])
    u = _sp(_mm(u, vw2[...]) + vb2[...])
    msg_out[...] = _mm(u, vw3[...]) + vb3[...]


def _edge_last_body(hs, hd, ea_in, ew1, eb1, ew2, eb2, ew3, eb3,
                    mw1, mb1, mw2, mb2, mw3, mb3, pe_out):
    c2 = hs[...] * hd[...]
    t = _sp(_mm(c2, ew1[...]) + eb1[...])
    t = _sp(_mm(t, ew2[...]) + eb2[...])
    ea = _mm(t, ew3[...]) + eb3[...] + ea_in[...]
    p = _sp(_mm(ea, mw1[...]) + mb1[...])
    p = _sp(_mm(p, mw2[...]) + mb2[...])
    pe_out[...] = jnp.transpose(_mm(p, mw3[...]) + mb3[...], (1, 0))


def _edge_step(kind, hs2, hd2, ea2, weight_list, ea_blk_off=0):
    e2 = hs2.shape[0]
    be = _E_BLK2
    body = {"first": _edge_first_body, "mid": _edge_mid_body,
            "last": _edge_last_body}[kind]
    w_specs = [pl.BlockSpec(w.shape, lambda i: (0, 0)) for w in weight_list]
    if kind == "last":
        out_specs = [pl.BlockSpec((2, be), lambda i: (0, i))]
        out_shape = [jax.ShapeDtypeStruct((2, e2), jnp.float32)]
    else:
        out_specs = [pl.BlockSpec((be, 64), lambda i: (i, 0)),
                     pl.BlockSpec((be, 128), lambda i: (i, 0))]
        out_shape = [jax.ShapeDtypeStruct((e2, 64), jnp.float32),
                     jax.ShapeDtypeStruct((e2, 128), jnp.float32)]
    return pl.pallas_call(
        body,
        grid=(e2 // be,),
        in_specs=[
            pl.BlockSpec((be, 128), lambda i: (i, 0)),
            pl.BlockSpec((be, 128), lambda i: (i, 0)),
            pl.BlockSpec((be, ea2.shape[1]), lambda i: (i + ea_blk_off, 0)),
        ] + w_specs,
        out_specs=out_specs,
        out_shape=out_shape,
    )(hs2, hd2, ea2, *weight_list)


def kernel(x, edge_attr, node_vel_emb, fa_params, fb_params, fe_params,
           fv_params, ke_params, mlp1_params, edge_index):
    n = x.shape[0]
    e = edge_attr.shape[0]
    src = edge_index[0]
    dst = edge_index[1]

    x2 = x.reshape(n // 2, 2 * x.shape[1])
    nv2 = node_vel_emb.reshape(n // 2, 2 * node_vel_emb.shape[1])
    eattr2 = edge_attr.reshape(e // 2, 2 * edge_attr.shape[1])

    h2, ke2 = _node_init(x2, nv2, fa_params, ke_params)

    (v1, b1), (v2, b2), (v3, b3) = fv_params
    fv_list = [_bd(v1[:64], 2), _bd(v1[64:], 2), _bt(b1, 2),
               _bd(v2, 2), _bt(b2, 2), _bd(v3, 2), _bt(b3, 2)]
    fb_list = [y for (w, b) in fb_params for y in (_bd(w, 2), _bt(b, 2))]
    fe_list = [y for (w, b) in fe_params for y in (_bd(w, 2), _bt(b, 2))]
    m1_list = [y for (w, b) in mlp1_params for y in (_bd(w, 2), _bt(b, 2))]

    eh = e // 2
    cb = 100
    srcs = [lax.slice(src, (0,), (eh,)), lax.slice(src, (eh,), (e,))]
    dsts = [lax.slice(dst, (0,), (eh,)), lax.slice(dst, (eh,), (e,))]
    dst3s = [d_.reshape(_NW, (eh // _NW) // cb, cb) for d_ in dsts]
    zeros = jnp.zeros((n // _NS, 64), jnp.float32)
    nblk_h = (eh // 2) // _E_BLK2

    eas = [eattr2, eattr2]
    ea_offs = [0, nblk_h]
    pes = [None, None]
    for step in range(3):
        h_lin = h2.reshape(n, 64)
        gath = [_sc_gather(h_lin, srcs[i], dsts[i]) for i in range(2)]
        msgs = [None, None]
        for i in range(2):
            hs2 = gath[i][0].reshape(eh // 2, 128)
            hd2 = gath[i][1].reshape(eh // 2, 128)
            if step == 0:
                eas[i], msgs[i] = _edge_step(
                    "first", hs2, hd2, eas[i], fb_list + fe_list + fv_list,
                    ea_blk_off=ea_offs[i])
            elif step == 1:
                eas[i], msgs[i] = _edge_step("mid", hs2, hd2, eas[i],
                                             fe_list + fv_list)
            else:
                pes[i] = _edge_step("last", hs2, hd2, eas[i],
                                    fe_list + m1_list)[0]
        if step < 2:
            parts = [_sc_scatter(msgs[i].reshape(eh, 64), dst3s[i], zeros, n)
                     for i in range(2)]
            h2 = _combine(h2, parts[0].reshape(2, n // 2, 128),
                          parts[1].reshape(2, n // 2, 128))

    pe = jnp.concatenate(
        [jnp.transpose(p_, (1, 0)).reshape(eh, 1) for p_ in pes], axis=0)
    ke_out = ke2.reshape(n, 1)
    return (pe, ke_out)
